# Initial kernel scaffold; baseline (speedup 1.0000x reference)
import functools

import jax
import jax.numpy as jnp
from jax import lax
from jax.experimental import pallas as pl
from jax.experimental.pallas import tpu as pltpu
from jax.experimental.pallas import tpu_sc as plsc

N = 10000
D = 128
E = 320000
NC = 2
NS = 16
NW = NC * NS
CHUNK = 128
CH = 4
BLK = CH * CHUNK
ROWS = E // CHUNK
RPAD = ((ROWS + NW - 1) // NW) * NW
EPAD = RPAD * CHUNK
RPW = RPAD // NW
NSP = 10016
ZR = NSP // NS
PAD_DST = N + 8

_mesh = plsc.VectorSubcoreMesh(
    core_axis_name="core", subcore_axis_name="subcore",
    num_cores=NC, num_subcores=NS)


def _sc_agg_body(with_deg, h_hbm, es_hbm, ed_hbm, *rest):
    if with_deg:
        part_hbm, degp_hbm, idx_s, idx_d, rows, ones16, agg_sp, deg_sp, sem = rest
    else:
        part_hbm, idx_s, idx_d, rows, ones16, agg_sp, deg_sp, sem = rest
        degp_hbm = None
    cidx = lax.axis_index("core")
    sidx = lax.axis_index("subcore")
    wid = cidx * NS + sidx

    zero = jnp.zeros((1, 16), jnp.float32)

    @pl.loop(0, CHUNK)
    def _(r):
        @pl.loop(0, D // 16)
        def _(c):
            rows[pl.ds(r, 1), pl.ds(c * 16, 16)] = zero

    @pl.loop(0, CHUNK)
    def _(r):
        ones16[pl.ds(r, 1), pl.ds(0, 16)] = zero

    base = sidx * ZR
    off = 0
    while off < ZR:
        n = min(CHUNK, ZR - off)
        pltpu.sync_copy(rows.at[pl.ds(0, n)], agg_sp.at[pl.ds(base + off, n)])
        if with_deg:
            pltpu.sync_copy(ones16.at[pl.ds(0, n)],
                            deg_sp.at[pl.ds(base + off, n)])
        off += n

    if with_deg:
        one = jnp.ones((1, 16), jnp.float32)

        @pl.loop(0, CHUNK)
        def _(r):
            ones16[pl.ds(r, 1), pl.ds(0, 16)] = one

    plsc.subcore_barrier()

    @pl.loop(0, RPW, step=CH)
    def _(b):
        row0 = wid * RPW + b
        pltpu.sync_copy(es_hbm.at[pl.ds(row0, CH)], idx_s)
        pltpu.sync_copy(ed_hbm.at[pl.ds(row0, CH)], idx_d)
        copies = []
        for k in range(CH):
            copies.append(pltpu.async_copy(
                h_hbm.at[idx_s.at[k]],
                rows.at[pl.ds(k * CHUNK, CHUNK)], sem))
        for k in range(CH):
            copies[k].wait()
        for k in range(CH):
            pltpu.sync_copy(rows.at[pl.ds(k * CHUNK, CHUNK)],
                            agg_sp.at[idx_d.at[k]], add=True)
        if with_deg:
            for k in range(CH):
                pltpu.sync_copy(ones16, deg_sp.at[idx_d.at[k]], add=True)

    plsc.subcore_barrier()

    pltpu.sync_copy(agg_sp.at[pl.ds(base, ZR)],
                    part_hbm.at[cidx].at[pl.ds(base, ZR)])
    if with_deg:
        pltpu.sync_copy(deg_sp.at[pl.ds(base, ZR)],
                        degp_hbm.at[cidx].at[pl.ds(base, ZR)])


def _make_sc_agg(with_deg):
    out_type = [jax.ShapeDtypeStruct((NC, NSP, D), jnp.float32)]
    if with_deg:
        out_type.append(jax.ShapeDtypeStruct((NC, NSP, 16), jnp.float32))
    return pl.kernel(
        functools.partial(_sc_agg_body, with_deg),
        out_type=tuple(out_type) if with_deg else out_type[0],
        mesh=_mesh,
        scratch_types=[
            pltpu.VMEM((CH, CHUNK), jnp.int32),
            pltpu.VMEM((CH, CHUNK), jnp.int32),
            pltpu.VMEM((BLK, D), jnp.float32),
            pltpu.VMEM((CHUNK, 16), jnp.float32),
            pltpu.VMEM_SHARED((NSP, D), jnp.float32),
            pltpu.VMEM_SHARED((NSP, 16), jnp.float32),
            pltpu.SemaphoreType.DMA,
        ],
    )


_sc_agg_deg = _make_sc_agg(True)
_sc_agg = _make_sc_agg(False)


def _tc_layer_body(relu, h_ref, p_ref, d_ref, w_ref, b_ref, o_ref):
    agg = p_ref[0, :N, :] + p_ref[1, :N, :]
    dsum = d_ref[0, :N, :] + d_ref[1, :N, :]
    deg = jnp.maximum(dsum[:, 0:1], 1.0)
    mean = agg / deg
    cat = jnp.concatenate([h_ref[...], mean], axis=1)
    y = lax.dot_general(cat, w_ref[...], (((1,), (0,)), ((), ())),
                        precision=lax.Precision.HIGHEST,
                        preferred_element_type=jnp.float32)
    y = y + b_ref[...]
    if relu:
        y = jnp.maximum(y, 0.0)
    o_ref[...] = y


def _tc_layer(h, part, degp, w, b, relu):
    return pl.pallas_call(
        functools.partial(_tc_layer_body, relu),
        out_shape=jax.ShapeDtypeStruct((N, D), jnp.float32),
    )(h, part, degp, w, b.reshape(1, D))


def kernel(x, edge_index, W0, b0, W1, b1):
    ei = edge_index.astype(jnp.int32)
    pad_s = jnp.zeros((EPAD - E,), jnp.int32)
    pad_d = jnp.full((EPAD - E,), PAD_DST, jnp.int32)
    es = jnp.concatenate([ei[0], pad_s]).reshape(RPAD, CHUNK)
    ed = jnp.concatenate([ei[1], pad_d]).reshape(RPAD, CHUNK)

    part0, degp = _sc_agg_deg(x, es, ed)
    h1 = _tc_layer(x, part0, degp, W0, b0, relu=True)
    part1 = _sc_agg(h1, es, ed)
    h2 = _tc_layer(h1, part1, degp, W1, b1, relu=False)
    return h2


# speedup vs baseline: 3.3830x; 3.3830x over previous
import functools

import jax
import jax.numpy as jnp
from jax import lax
from jax.experimental import pallas as pl
from jax.experimental.pallas import tpu as pltpu
from jax.experimental.pallas import tpu_sc as plsc

N = 10000
D = 128
E = 320000
NC = 2
NS = 16
NW = NC * NS
CHUNK = 128
CH = 2
BLK = CH * CHUNK
ROWS = E // CHUNK
GR = 8
RPAD = ((ROWS + NW * GR - 1) // (NW * GR)) * (NW * GR)
EPAD = RPAD * CHUNK
RPW = RPAD // NW
NSP = 10112
ZR = NSP // NS
PAD_DST = N + 8

_mesh = plsc.VectorSubcoreMesh(
    core_axis_name="core", subcore_axis_name="subcore",
    num_cores=NC, num_subcores=NS)


def _zero_rows(ref, nrows, width):
    zero = jnp.zeros((1, 16), jnp.float32)

    @pl.loop(0, nrows)
    def _(r):
        @pl.loop(0, width // 16)
        def _(c):
            ref[pl.ds(r, 1), pl.ds(c * 16, 16)] = zero


def _zero_shared_slice(src, dst, base, total):
    off = 0
    while off < total:
        n = min(CHUNK, total - off)
        pltpu.sync_copy(src.at[pl.ds(0, n)], dst.at[pl.ds(base + off, n)])
        off += n


def _sc_agg_body(h_hbm, es_hbm, ed_hbm, part_hbm, idx_s, idx_d, rows,
                 agg_sp, sem):
    cidx = lax.axis_index("core")
    sidx = lax.axis_index("subcore")
    wid = cidx * NS + sidx

    _zero_rows(rows, CHUNK, D)
    base = sidx * ZR
    _zero_shared_slice(rows, agg_sp, base, ZR)
    plsc.subcore_barrier()

    @pl.loop(0, RPW, step=GR)
    def _(b):
        row0 = wid * RPW + b
        pltpu.sync_copy(es_hbm.at[pl.ds(row0, GR)], idx_s)
        pltpu.sync_copy(ed_hbm.at[pl.ds(row0, GR)], idx_d)
        for w in range(GR // CH):
            copies = []
            for j in range(CH):
                copies.append(pltpu.async_copy(
                    h_hbm.at[idx_s.at[w * CH + j]],
                    rows.at[pl.ds(j * CHUNK, CHUNK)], sem))
            for j in range(CH):
                copies[j].wait()
            for j in range(CH):
                pltpu.sync_copy(rows.at[pl.ds(j * CHUNK, CHUNK)],
                                agg_sp.at[idx_d.at[w * CH + j]], add=True)

    plsc.subcore_barrier()
    pltpu.sync_copy(agg_sp.at[pl.ds(base, ZR)],
                    part_hbm.at[cidx].at[pl.ds(base, ZR)])


_sc_agg = pl.kernel(
    _sc_agg_body,
    out_type=jax.ShapeDtypeStruct((NC, NSP, D), jnp.float32),
    mesh=_mesh,
    scratch_types=[
        pltpu.VMEM((GR, CHUNK), jnp.int32),
        pltpu.VMEM((GR, CHUNK), jnp.int32),
        pltpu.VMEM((BLK, D), jnp.float32),
        pltpu.VMEM_SHARED((NSP, D), jnp.float32),
        pltpu.SemaphoreType.DMA,
    ],
)


def _sc_deg_body(ed_hbm, degp_hbm, idx_d, ones_b, deg_sp):
    cidx = lax.axis_index("core")
    sidx = lax.axis_index("subcore")
    wid = cidx * NS + sidx

    _zero_rows(ones_b, CHUNK, D)
    base = sidx * ZR
    _zero_shared_slice(ones_b, deg_sp, base, ZR)
    one = jnp.ones((1, 16), jnp.float32)

    @pl.loop(0, CHUNK)
    def _(r):
        @pl.loop(0, D // 16)
        def _(c):
            ones_b[pl.ds(r, 1), pl.ds(c * 16, 16)] = one

    plsc.subcore_barrier()

    @pl.loop(0, RPW, step=GR)
    def _(b):
        row0 = wid * RPW + b
        pltpu.sync_copy(ed_hbm.at[pl.ds(row0, GR)], idx_d)
        for k in range(GR):
            pltpu.sync_copy(ones_b, deg_sp.at[idx_d.at[k]], add=True)

    plsc.subcore_barrier()
    pltpu.sync_copy(deg_sp.at[pl.ds(base, ZR)],
                    degp_hbm.at[cidx].at[pl.ds(base, ZR)])


_sc_deg = pl.kernel(
    _sc_deg_body,
    out_type=jax.ShapeDtypeStruct((NC, NSP, D), jnp.float32),
    mesh=_mesh,
    scratch_types=[
        pltpu.VMEM((GR, CHUNK), jnp.int32),
        pltpu.VMEM((CHUNK, D), jnp.float32),
        pltpu.VMEM_SHARED((NSP, D), jnp.float32),
    ],
)


RB = 1000


def _tc_layer_body(relu, h_ref, p_ref, d_ref, w_ref, b_ref, o_ref):
    agg = p_ref[0] + p_ref[1]
    dsum = d_ref[0] + d_ref[1]
    deg = jnp.maximum(dsum[:, 0:1], 1.0)
    mean = agg / deg
    cat = jnp.concatenate([h_ref[...], mean], axis=1)
    y = lax.dot_general(cat, w_ref[...], (((1,), (0,)), ((), ())),
                        precision=lax.Precision.HIGHEST,
                        preferred_element_type=jnp.float32)
    y = y + b_ref[...]
    if relu:
        y = jnp.maximum(y, 0.0)
    o_ref[...] = y


def _tc_layer(h, part, degp, w, b, relu):
    return pl.pallas_call(
        functools.partial(_tc_layer_body, relu),
        grid=(N // RB,),
        in_specs=[
            pl.BlockSpec((RB, D), lambda i: (i, 0)),
            pl.BlockSpec((NC, RB, D), lambda i: (0, i, 0)),
            pl.BlockSpec((NC, RB, D), lambda i: (0, i, 0)),
            pl.BlockSpec((2 * D, D), lambda i: (0, 0)),
            pl.BlockSpec((1, D), lambda i: (0, 0)),
        ],
        out_specs=pl.BlockSpec((RB, D), lambda i: (i, 0)),
        out_shape=jax.ShapeDtypeStruct((N, D), jnp.float32),
    )(h, part, degp, w, b.reshape(1, D))


def kernel(x, edge_index, W0, b0, W1, b1):
    ei = edge_index.astype(jnp.int32)
    pad_s = jnp.zeros((EPAD - E,), jnp.int32)
    pad_d = jnp.full((EPAD - E,), PAD_DST, jnp.int32)
    es = jnp.concatenate([ei[0], pad_s]).reshape(RPAD, CHUNK)
    ed = jnp.concatenate([ei[1], pad_d]).reshape(RPAD, CHUNK)

    degp = _sc_deg(ed)
    part0 = _sc_agg(x, es, ed)
    h1 = _tc_layer(x, part0, degp, W0, b0, relu=True)
    part1 = _sc_agg(h1, es, ed)
    h2 = _tc_layer(h1, part1, degp, W1, b1, relu=False)
    return h2


# speedup vs baseline: 8.3429x; 2.4661x over previous
import functools

import jax
import jax.numpy as jnp
from jax import lax
from jax.experimental import pallas as pl
from jax.experimental.pallas import tpu as pltpu
from jax.experimental.pallas import tpu_sc as plsc

N = 10000
D = 128
E = 320000
NC = 2
NS = 16
NW = NC * NS
CHUNK = 128
CH = 2
BLK = CH * CHUNK
ROWS = E // CHUNK
GR = 8
RPAD = ((ROWS + NW * GR - 1) // (NW * GR)) * (NW * GR)
EPAD = RPAD * CHUNK
RPW = RPAD // NW
NSP = 10112
ZR = NSP // NS
PAD_DST = N + 8

_mesh = plsc.VectorSubcoreMesh(
    core_axis_name="core", subcore_axis_name="subcore",
    num_cores=NC, num_subcores=NS)


def _zero_rows(ref, nrows, width):
    zero = jnp.zeros((1, 16), jnp.float32)

    @pl.loop(0, nrows)
    def _(r):
        @pl.loop(0, width // 16)
        def _(c):
            ref[pl.ds(r, 1), pl.ds(c * 16, 16)] = zero


def _zero_shared_slice(src, dst, base, total):
    off = 0
    while off < total:
        n = min(CHUNK, total - off)
        pltpu.sync_copy(src.at[pl.ds(0, n)], dst.at[pl.ds(base + off, n)])
        off += n


def _sc_agg_body(h_hbm, es_hbm, ed_hbm, part_hbm, idx_s, idx_d, rows,
                 agg_sp, sem):
    cidx = lax.axis_index("core")
    sidx = lax.axis_index("subcore")
    wid = cidx * NS + sidx

    _zero_rows(rows, CHUNK, D)
    base = sidx * ZR
    _zero_shared_slice(rows, agg_sp, base, ZR)
    plsc.subcore_barrier()

    @pl.loop(0, RPW, step=GR)
    def _(b):
        row0 = wid * RPW + b
        pltpu.sync_copy(es_hbm.at[pl.ds(row0, GR)], idx_s)
        pltpu.sync_copy(ed_hbm.at[pl.ds(row0, GR)], idx_d)
        for w in range(GR // CH):
            copies = []
            for j in range(CH):
                copies.append(pltpu.async_copy(
                    h_hbm.at[idx_s.at[w * CH + j]],
                    rows.at[pl.ds(j * CHUNK, CHUNK)], sem))
            for j in range(CH):
                copies[j].wait()
            for j in range(CH):
                pltpu.sync_copy(rows.at[pl.ds(j * CHUNK, CHUNK)],
                                agg_sp.at[idx_d.at[w * CH + j]], add=True)

    plsc.subcore_barrier()
    pltpu.sync_copy(agg_sp.at[pl.ds(base, ZR)],
                    part_hbm.at[cidx].at[pl.ds(base, ZR)])


_sc_agg = pl.kernel(
    _sc_agg_body,
    out_type=jax.ShapeDtypeStruct((NC, NSP, D), jnp.float32),
    mesh=_mesh,
    scratch_types=[
        pltpu.VMEM((GR, CHUNK), jnp.int32),
        pltpu.VMEM((GR, CHUNK), jnp.int32),
        pltpu.VMEM((BLK, D), jnp.float32),
        pltpu.VMEM_SHARED((NSP, D), jnp.float32),
        pltpu.SemaphoreType.DMA,
    ],
)


def _sc_deg_body(ed_hbm, degp_hbm, idx_d, ones_b, deg_sp):
    cidx = lax.axis_index("core")
    sidx = lax.axis_index("subcore")
    wid = cidx * NS + sidx

    _zero_rows(ones_b, CHUNK, D)
    base = sidx * ZR
    _zero_shared_slice(ones_b, deg_sp, base, ZR)
    one = jnp.ones((1, 16), jnp.float32)

    @pl.loop(0, CHUNK)
    def _(r):
        @pl.loop(0, D // 16)
        def _(c):
            ones_b[pl.ds(r, 1), pl.ds(c * 16, 16)] = one

    plsc.subcore_barrier()

    @pl.loop(0, RPW, step=GR)
    def _(b):
        row0 = wid * RPW + b
        pltpu.sync_copy(ed_hbm.at[pl.ds(row0, GR)], idx_d)
        for k in range(GR):
            pltpu.sync_copy(ones_b, deg_sp.at[idx_d.at[k]], add=True)

    plsc.subcore_barrier()
    pltpu.sync_copy(deg_sp.at[pl.ds(base, ZR)],
                    degp_hbm.at[cidx].at[pl.ds(base, ZR)])


_sc_deg = pl.kernel(
    _sc_deg_body,
    out_type=jax.ShapeDtypeStruct((NC, NSP, D), jnp.float32),
    mesh=_mesh,
    scratch_types=[
        pltpu.VMEM((GR, CHUNK), jnp.int32),
        pltpu.VMEM((CHUNK, D), jnp.float32),
        pltpu.VMEM_SHARED((NSP, D), jnp.float32),
    ],
)


RB = 1000


def _tc_layer_body(relu, h_ref, p_ref, d_ref, w_ref, b_ref, o_ref):
    agg = p_ref[0] + p_ref[1]
    dsum = d_ref[0] + d_ref[1]
    deg = jnp.maximum(dsum[:, 0:1], 1.0)
    mean = agg / deg
    cat = jnp.concatenate([h_ref[...], mean], axis=1)
    y = lax.dot_general(cat, w_ref[...], (((1,), (0,)), ((), ())),
                        precision=lax.Precision.HIGHEST,
                        preferred_element_type=jnp.float32)
    y = y + b_ref[...]
    if relu:
        y = jnp.maximum(y, 0.0)
    o_ref[...] = y


def _tc_layer(h, part, degp, w, b, relu):
    return pl.pallas_call(
        functools.partial(_tc_layer_body, relu),
        grid=(N // RB,),
        in_specs=[
            pl.BlockSpec((RB, D), lambda i: (i, 0)),
            pl.BlockSpec((NC, RB, D), lambda i: (0, i, 0)),
            pl.BlockSpec((NC, RB, D), lambda i: (0, i, 0)),
            pl.BlockSpec((2 * D, D), lambda i: (0, 0)),
            pl.BlockSpec((1, D), lambda i: (0, 0)),
        ],
        out_specs=pl.BlockSpec((RB, D), lambda i: (i, 0)),
        out_shape=jax.ShapeDtypeStruct((N, D), jnp.float32),
    )(h, part, degp, w, b.reshape(1, D))


def kernel(x, edge_index, W0, b0, W1, b1):
    ei = edge_index.astype(jnp.int32)
    r = jnp.arange(EPAD - E, dtype=jnp.int32)
    pad_s = r % N
    pad_d = N + (r % (NSP - N))
    es = jnp.concatenate([ei[0], pad_s]).reshape(RPAD, CHUNK)
    ed = jnp.concatenate([ei[1], pad_d]).reshape(RPAD, CHUNK)

    degp = _sc_deg(ed)
    part0 = _sc_agg(x, es, ed)
    h1 = _tc_layer(x, part0, degp, W0, b0, relu=True)
    part1 = _sc_agg(h1, es, ed)
    h2 = _tc_layer(h1, part1, degp, W1, b1, relu=False)
    return h2


# speedup vs baseline: 10.2943x; 1.2339x over previous
import functools

import jax
import jax.numpy as jnp
from jax import lax
from jax.experimental import pallas as pl
from jax.experimental.pallas import tpu as pltpu
from jax.experimental.pallas import tpu_sc as plsc

N = 10000
D = 128
E = 320000
NC = 2
NS = 16
NW = NC * NS
CHUNK = 128
NSLOT = 2
ROWS = E // CHUNK
GR = 8
RPAD = ((ROWS + NW * GR - 1) // (NW * GR)) * (NW * GR)
RPF = RPAD + 2 * GR
EPAD = RPAD * CHUNK
RPW = RPAD // NW
NBLK = RPW // GR
NSP = 10112
ZR = NSP // NS

_mesh = plsc.VectorSubcoreMesh(
    core_axis_name="core", subcore_axis_name="subcore",
    num_cores=NC, num_subcores=NS)


def _zero_rows(ref, nrows, width):
    zero = jnp.zeros((1, 16), jnp.float32)

    @pl.loop(0, nrows)
    def _(r):
        @pl.loop(0, width // 16)
        def _(c):
            ref[pl.ds(r, 1), pl.ds(c * 16, 16)] = zero


def _zero_shared_slice(src, dst, base, total):
    off = 0
    while off < total:
        n = min(CHUNK, total - off)
        pltpu.sync_copy(src.at[pl.ds(0, n)], dst.at[pl.ds(base + off, n)])
        off += n


def _sc_agg_body(h_hbm, es_hbm, ed_hbm, part_hbm, ibs0, ibd0, ibs1, ibd1,
                 rows, agg_sp, sem_i0, sem_i1, sem_g0, sem_g1, sem_s0,
                 sem_s1):
    cidx = lax.axis_index("core")
    sidx = lax.axis_index("subcore")
    wid = cidx * NS + sidx
    wbase = wid * RPW

    _zero_rows(rows, CHUNK, D)
    base = sidx * ZR
    _zero_shared_slice(rows, agg_sp, base, ZR)
    plsc.subcore_barrier()

    sem_g = (sem_g0, sem_g1)
    sem_s = (sem_s0, sem_s1)

    def run_block(row0, ibs, ibd):
        hg = {}
        hs = {}
        hg[0] = pltpu.async_copy(
            h_hbm.at[ibs.at[0]], rows.at[pl.ds(0, CHUNK)], sem_g[0])
        for k in range(GR):
            slot = k % NSLOT
            if k + 1 < GR:
                nslot = (k + 1) % NSLOT
                if k - 1 >= 0:
                    hs[k - 1].wait()
                hg[k + 1] = pltpu.async_copy(
                    h_hbm.at[ibs.at[k + 1]],
                    rows.at[pl.ds(nslot * CHUNK, CHUNK)], sem_g[nslot])
            hg[k].wait()
            hs[k] = pltpu.async_copy(
                rows.at[pl.ds(slot * CHUNK, CHUNK)],
                agg_sp.at[ibd.at[k]], sem_s[slot], add=True)
        hs[GR - 2].wait()
        hs[GR - 1].wait()

    pltpu.sync_copy(es_hbm.at[pl.ds(wbase, GR)], ibs0)
    pltpu.sync_copy(ed_hbm.at[pl.ds(wbase, GR)], ibd0)
    pltpu.async_copy(es_hbm.at[pl.ds(wbase + GR, GR)], ibs1, sem_i1)
    pltpu.async_copy(ed_hbm.at[pl.ds(wbase + GR, GR)], ibd1, sem_i1)

    @pl.loop(0, NBLK, step=2)
    def _(b):
        row0 = wbase + b * GR

        @pl.when(b > 0)
        def _():
            pltpu.make_async_copy(es_hbm.at[pl.ds(row0, GR)], ibs0,
                                  sem_i0).wait()
            pltpu.make_async_copy(ed_hbm.at[pl.ds(row0, GR)], ibd0,
                                  sem_i0).wait()

        run_block(row0, ibs0, ibd0)
        pltpu.async_copy(es_hbm.at[pl.ds(row0 + 2 * GR, GR)], ibs0, sem_i0)
        pltpu.async_copy(ed_hbm.at[pl.ds(row0 + 2 * GR, GR)], ibd0, sem_i0)

        row1 = row0 + GR
        pltpu.make_async_copy(es_hbm.at[pl.ds(row1, GR)], ibs1, sem_i1).wait()
        pltpu.make_async_copy(ed_hbm.at[pl.ds(row1, GR)], ibd1, sem_i1).wait()
        run_block(row1, ibs1, ibd1)
        pltpu.async_copy(es_hbm.at[pl.ds(row1 + 2 * GR, GR)], ibs1, sem_i1)
        pltpu.async_copy(ed_hbm.at[pl.ds(row1 + 2 * GR, GR)], ibd1, sem_i1)

    pltpu.make_async_copy(es_hbm.at[pl.ds(wbase, GR)], ibs0, sem_i0).wait()
    pltpu.make_async_copy(ed_hbm.at[pl.ds(wbase, GR)], ibd0, sem_i0).wait()
    pltpu.make_async_copy(es_hbm.at[pl.ds(wbase, GR)], ibs1, sem_i1).wait()
    pltpu.make_async_copy(ed_hbm.at[pl.ds(wbase, GR)], ibd1, sem_i1).wait()

    plsc.subcore_barrier()
    pltpu.sync_copy(agg_sp.at[pl.ds(base, ZR)],
                    part_hbm.at[cidx].at[pl.ds(base, ZR)])


_sc_agg = pl.kernel(
    _sc_agg_body,
    out_type=jax.ShapeDtypeStruct((NC, NSP, D), jnp.float32),
    mesh=_mesh,
    scratch_types=[
        pltpu.VMEM((GR, CHUNK), jnp.int32),
        pltpu.VMEM((GR, CHUNK), jnp.int32),
        pltpu.VMEM((GR, CHUNK), jnp.int32),
        pltpu.VMEM((GR, CHUNK), jnp.int32),
        pltpu.VMEM((NSLOT * CHUNK, D), jnp.float32),
        pltpu.VMEM_SHARED((NSP, D), jnp.float32),
        pltpu.SemaphoreType.DMA,
        pltpu.SemaphoreType.DMA,
        pltpu.SemaphoreType.DMA,
        pltpu.SemaphoreType.DMA,
        pltpu.SemaphoreType.DMA,
        pltpu.SemaphoreType.DMA,
    ],
)


def _sc_deg_body(ed_hbm, degp_hbm, ibd0, ibd1, ones_b, deg_sp, sem_i0,
                 sem_i1, sem_s):
    cidx = lax.axis_index("core")
    sidx = lax.axis_index("subcore")
    wid = cidx * NS + sidx
    wbase = wid * RPW

    _zero_rows(ones_b, CHUNK, D)
    base = sidx * ZR
    _zero_shared_slice(ones_b, deg_sp, base, ZR)
    one = jnp.ones((1, 16), jnp.float32)

    @pl.loop(0, CHUNK)
    def _(r):
        @pl.loop(0, D // 16)
        def _(c):
            ones_b[pl.ds(r, 1), pl.ds(c * 16, 16)] = one

    plsc.subcore_barrier()

    def run_block(ibd):
        hs = []
        for k in range(GR):
            hs.append(pltpu.async_copy(ones_b, deg_sp.at[ibd.at[k]], sem_s,
                                       add=True))
        for h in hs:
            h.wait()

    pltpu.sync_copy(ed_hbm.at[pl.ds(wbase, GR)], ibd0)
    pltpu.async_copy(ed_hbm.at[pl.ds(wbase + GR, GR)], ibd1, sem_i1)

    @pl.loop(0, NBLK, step=2)
    def _(b):
        row0 = wbase + b * GR

        @pl.when(b > 0)
        def _():
            pltpu.make_async_copy(ed_hbm.at[pl.ds(row0, GR)], ibd0,
                                  sem_i0).wait()

        run_block(ibd0)
        pltpu.async_copy(ed_hbm.at[pl.ds(row0 + 2 * GR, GR)], ibd0, sem_i0)

        row1 = row0 + GR
        pltpu.make_async_copy(ed_hbm.at[pl.ds(row1, GR)], ibd1, sem_i1).wait()
        run_block(ibd1)
        pltpu.async_copy(ed_hbm.at[pl.ds(row1 + 2 * GR, GR)], ibd1, sem_i1)

    pltpu.make_async_copy(ed_hbm.at[pl.ds(wbase, GR)], ibd0, sem_i0).wait()
    pltpu.make_async_copy(ed_hbm.at[pl.ds(wbase, GR)], ibd1, sem_i1).wait()

    plsc.subcore_barrier()
    pltpu.sync_copy(deg_sp.at[pl.ds(base, ZR)],
                    degp_hbm.at[cidx].at[pl.ds(base, ZR)])


_sc_deg = pl.kernel(
    _sc_deg_body,
    out_type=jax.ShapeDtypeStruct((NC, NSP, D), jnp.float32),
    mesh=_mesh,
    scratch_types=[
        pltpu.VMEM((GR, CHUNK), jnp.int32),
        pltpu.VMEM((GR, CHUNK), jnp.int32),
        pltpu.VMEM((CHUNK, D), jnp.float32),
        pltpu.VMEM_SHARED((NSP, D), jnp.float32),
        pltpu.SemaphoreType.DMA,
        pltpu.SemaphoreType.DMA,
        pltpu.SemaphoreType.DMA,
    ],
)


RB = 1000


def _tc_layer_body(relu, h_ref, p_ref, d_ref, w_ref, b_ref, o_ref):
    agg = p_ref[0] + p_ref[1]
    dsum = d_ref[0] + d_ref[1]
    deg = jnp.maximum(dsum[:, 0:1], 1.0)
    mean = agg / deg
    cat = jnp.concatenate([h_ref[...], mean], axis=1)
    y = lax.dot_general(cat, w_ref[...], (((1,), (0,)), ((), ())),
                        precision=lax.Precision.HIGHEST,
                        preferred_element_type=jnp.float32)
    y = y + b_ref[...]
    if relu:
        y = jnp.maximum(y, 0.0)
    o_ref[...] = y


def _tc_layer(h, part, degp, w, b, relu):
    return pl.pallas_call(
        functools.partial(_tc_layer_body, relu),
        grid=(N // RB,),
        in_specs=[
            pl.BlockSpec((RB, D), lambda i: (i, 0)),
            pl.BlockSpec((NC, RB, D), lambda i: (0, i, 0)),
            pl.BlockSpec((NC, RB, D), lambda i: (0, i, 0)),
            pl.BlockSpec((2 * D, D), lambda i: (0, 0)),
            pl.BlockSpec((1, D), lambda i: (0, 0)),
        ],
        out_specs=pl.BlockSpec((RB, D), lambda i: (i, 0)),
        out_shape=jax.ShapeDtypeStruct((N, D), jnp.float32),
    )(h, part, degp, w, b.reshape(1, D))


def kernel(x, edge_index, W0, b0, W1, b1):
    ei = edge_index.astype(jnp.int32)
    r = jnp.arange(RPF * CHUNK - E, dtype=jnp.int32)
    pad_s = r % N
    pad_d = N + (r % (NSP - N))
    es = jnp.concatenate([ei[0], pad_s]).reshape(RPF, CHUNK)
    ed = jnp.concatenate([ei[1], pad_d]).reshape(RPF, CHUNK)

    degp = _sc_deg(ed)
    part0 = _sc_agg(x, es, ed)
    h1 = _tc_layer(x, part0, degp, W0, b0, relu=True)
    part1 = _sc_agg(h1, es, ed)
    h2 = _tc_layer(h1, part1, degp, W1, b1, relu=False)
    return h2
